# baseline (speedup 1.0000x reference)
import functools

import jax
import jax.numpy as jnp
from jax import lax
from jax.experimental import pallas as pl
from jax.experimental.pallas import tpu as pltpu
from jax.experimental.pallas import tpu_sc as plsc

EMB = 64
LSEQ = 200
NC = 2
NS = 16
NW = NC * NS
C = 512


def _rsqrt(a):
    i = plsc.bitcast(a, jnp.int32)
    i = jnp.int32(0x5F3759DF) - (i >> 1)
    y = plsc.bitcast(i, jnp.float32)
    for _ in range(3):
        y = y * (1.5 - 0.5 * a * y * y)
    return y


def _make_kernel(N):
    per_w = N // NW
    nchunks = per_w // C
    assert nchunks % 2 == 0
    mesh = plsc.VectorSubcoreMesh(core_axis_name="c", subcore_axis_name="s")

    @functools.partial(
        pl.kernel,
        out_type=jax.ShapeDtypeStruct((N, EMB), jnp.float32),
        mesh=mesh,
        compiler_params=pltpu.CompilerParams(
            needs_layout_passes=False, use_tc_tiling_on_sc=False),
        scratch_types=[
            pltpu.VMEM((C,), jnp.int32),
            pltpu.VMEM((C,), jnp.int32),
            pltpu.VMEM((C,), jnp.int32),
            pltpu.VMEM((C,), jnp.int32),
            pltpu.VMEM((C, EMB), jnp.float32),
            pltpu.VMEM((C, EMB), jnp.float32),
            pltpu.VMEM((C, EMB), jnp.float32),
            pltpu.VMEM((LSEQ, EMB), jnp.float32),
            pltpu.VMEM((3, EMB), jnp.float32),
            pltpu.VMEM((EMB,), jnp.float32),
            pltpu.VMEM((EMB,), jnp.float32),
            pltpu.SemaphoreType.DMA,
            pltpu.SemaphoreType.DMA,
            pltpu.SemaphoreType.DMA,
            pltpu.SemaphoreType.DMA,
            pltpu.SemaphoreType.DMA,
        ],
    )
    def body(tok_hbm, mrk_hbm, tbl_hbm, tim_hbm, sen_hbm, g_hbm, b_hbm,
             out_hbm, idx0, idx1, mrk0, mrk1, rows0, rows1, ybuf,
             tim_v, sen_v, g_v, b_v, gs0, gs1, osem, is0, is1):
        wid = lax.axis_index("s") * NC + lax.axis_index("c")
        base = wid * per_w
        idx_b = [idx0, idx1]
        mrk_b = [mrk0, mrk1]
        rows_b = [rows0, rows1]
        gs_b = [gs0, gs1]
        is_b = [is0, is1]
        pltpu.sync_copy(tim_hbm, tim_v)
        pltpu.sync_copy(sen_hbm, sen_v)
        pltpu.sync_copy(g_hbm, g_v)
        pltpu.sync_copy(b_hbm, b_v)
        iota16 = lax.iota(jnp.int32, 16)

        pltpu.sync_copy(tok_hbm.at[pl.ds(base, C)], idx_b[0])
        pltpu.sync_copy(mrk_hbm.at[pl.ds(base, C)], mrk_b[0])
        pltpu.async_copy(tbl_hbm.at[idx_b[0]], rows_b[0], gs_b[0])
        pltpu.async_copy(tok_hbm.at[pl.ds(base + C, C)], idx_b[1], is_b[1])
        pltpu.async_copy(mrk_hbm.at[pl.ds(base + C, C)], mrk_b[1], is_b[1])

        def compute_chunk(off, rows_v, idx_v, mrk_v):
            @plsc.parallel_loop(0, C, step=16)
            def group(r0):
                ids = idx_v[pl.ds(r0, 16)]
                mrk = mrk_v[pl.ds(r0, 16)]
                rows16 = r0 + iota16
                pred = ids != 0
                sj = jnp.where(mrk == 3, 0, mrk)
                w = (off + r0) % LSEQ + iota16
                lj = jnp.where(w >= LSEQ, w - LSEQ, w)

                def load_x(c):
                    cc = jnp.full((16,), c, jnp.int32)
                    tok = plsc.load_gather(rows_v, [rows16, cc])
                    tim = plsc.load_gather(tim_v, [lj, cc])
                    sen = plsc.load_gather(sen_v, [sj, cc])
                    return cc, tok + jnp.where(pred, tim, 0.0) + sen

                s1 = jnp.zeros((16,), jnp.float32)
                s2 = jnp.zeros((16,), jnp.float32)
                for c in range(EMB):
                    cc, xc = load_x(c)
                    s1 = s1 + xc
                    s2 = s2 + xc * xc
                    plsc.store_scatter(rows_v, [rows16, cc], xc)
                mu = s1 * (1.0 / EMB)
                var = s2 * (1.0 / EMB) - mu * mu
                rs = _rsqrt(var + 1e-5)
                for c in range(EMB):
                    cc = jnp.full((16,), c, jnp.int32)
                    xc = plsc.load_gather(rows_v, [rows16, cc])
                    gc = plsc.load_gather(g_v, [cc])
                    bc = plsc.load_gather(b_v, [cc])
                    y = (xc - mu) * rs * gc + bc
                    plsc.store_scatter(ybuf, [rows16, cc], y)

        def iter_body(i2, carry):
            for b in range(2):
                i = i2 * 2 + b
                off = base + i * C
                nb = 1 - b

                @pl.when(i + 1 < nchunks)
                def _():
                    pltpu.make_async_copy(
                        tok_hbm.at[pl.ds(off + C, C)], idx_b[nb],
                        is_b[nb]).wait()
                    pltpu.make_async_copy(
                        mrk_hbm.at[pl.ds(off + C, C)], mrk_b[nb],
                        is_b[nb]).wait()
                    pltpu.async_copy(tbl_hbm.at[idx_b[nb]], rows_b[nb],
                                     gs_b[nb])

                pltpu.make_async_copy(tbl_hbm.at[idx_b[b]], rows_b[b],
                                      gs_b[b]).wait()

                @pl.when(i >= 1)
                def _():
                    pltpu.make_async_copy(
                        ybuf, out_hbm.at[pl.ds(off - C, C)], osem).wait()

                compute_chunk(off, rows_b[b], idx_b[b], mrk_b[b])
                pltpu.async_copy(ybuf, out_hbm.at[pl.ds(off, C)], osem)

                @pl.when(i + 2 < nchunks)
                def _():
                    offn2 = off + 2 * C
                    pltpu.async_copy(tok_hbm.at[pl.ds(offn2, C)], idx_b[b],
                                     is_b[b])
                    pltpu.async_copy(mrk_hbm.at[pl.ds(offn2, C)], mrk_b[b],
                                     is_b[b])
            return carry

        lax.fori_loop(0, nchunks // 2, iter_body, 0)
        pltpu.make_async_copy(
            ybuf, out_hbm.at[pl.ds(base + (nchunks - 1) * C, C)],
            osem).wait()

    return body


def kernel(batTok, tokMrk, tokEmbTbl, timEmbTbl, senEmbTbl, gamma, beta):
    B, L = batTok.shape
    N = B * L
    tok_flat = batTok.reshape(N).astype(jnp.int32)
    mrk_flat = tokMrk.reshape(N).astype(jnp.int32)
    out = _make_kernel(N)(
        tok_flat, mrk_flat,
        tokEmbTbl.astype(jnp.float32),
        timEmbTbl.astype(jnp.float32),
        senEmbTbl.astype(jnp.float32),
        gamma.astype(jnp.float32),
        beta.astype(jnp.float32),
    )
    return out.reshape(B, L, EMB)


# speedup vs baseline: 3.2548x; 3.2548x over previous
import functools

import jax
import jax.numpy as jnp
from jax import lax
from jax.experimental import pallas as pl
from jax.experimental.pallas import tpu as pltpu
from jax.experimental.pallas import tpu_sc as plsc

EMB = 64
LSEQ = 200
NC = 2
NS = 16
NW = NC * NS
C = 512


def _rsqrt(a):
    i = plsc.bitcast(a, jnp.int32)
    i = jnp.int32(0x5F3759DF) - (i >> 1)
    y = plsc.bitcast(i, jnp.float32)
    for _ in range(3):
        y = y * (1.5 - 0.5 * a * y * y)
    return y


def _make_kernel(N):
    per_w = N // NW
    nchunks = per_w // C
    assert nchunks % 2 == 0
    mesh = plsc.VectorSubcoreMesh(core_axis_name="c", subcore_axis_name="s")

    @functools.partial(
        pl.kernel,
        out_type=jax.ShapeDtypeStruct((N, EMB), jnp.float32),
        mesh=mesh,
        compiler_params=pltpu.CompilerParams(
            needs_layout_passes=False, use_tc_tiling_on_sc=False),
        scratch_types=[
            pltpu.VMEM((C,), jnp.int32),
            pltpu.VMEM((C,), jnp.int32),
            pltpu.VMEM((C,), jnp.int32),
            pltpu.VMEM((C,), jnp.int32),
            pltpu.VMEM((C, EMB), jnp.float32),
            pltpu.VMEM((C, EMB), jnp.float32),
            pltpu.VMEM((LSEQ, EMB), jnp.float32),
            pltpu.VMEM((3, EMB), jnp.float32),
            pltpu.VMEM(((LSEQ + 1) * 3, EMB), jnp.float32),
            pltpu.VMEM((EMB,), jnp.float32),
            pltpu.VMEM((EMB,), jnp.float32),
            pltpu.SemaphoreType.DMA,
            pltpu.SemaphoreType.DMA,
            pltpu.SemaphoreType.DMA,
            pltpu.SemaphoreType.DMA,
            pltpu.SemaphoreType.DMA,
            pltpu.SemaphoreType.DMA,
        ],
    )
    def body(tok_hbm, mrk_hbm, tbl_hbm, tim_hbm, sen_hbm, g_hbm, b_hbm,
             out_hbm, idx0, idx1, mrk0, mrk1, rows0, rows1,
             tim_v, sen_v, ts_v, g_v, b_v, gs0, gs1, os0, os1, is0, is1):
        wid = lax.axis_index("s") * NC + lax.axis_index("c")
        base = wid * per_w
        idx_b = [idx0, idx1]
        mrk_b = [mrk0, mrk1]
        rows_b = [rows0, rows1]
        gs_b = [gs0, gs1]
        os_b = [os0, os1]
        is_b = [is0, is1]
        pltpu.sync_copy(tim_hbm, tim_v)
        pltpu.sync_copy(sen_hbm, sen_v)
        pltpu.sync_copy(g_hbm, g_v)
        pltpu.sync_copy(b_hbm, b_v)
        nk = EMB // 16
        g_k = [g_v[pl.ds(k * 16, 16)] for k in range(nk)]
        b_k = [b_v[pl.ds(k * 16, 16)] for k in range(nk)]

        def build_ts(l, carry):
            for s in range(3):
                for k in range(nk):
                    ts_v[l * 3 + s, pl.ds(k * 16, 16)] = (
                        tim_v[l, pl.ds(k * 16, 16)]
                        + sen_v[s, pl.ds(k * 16, 16)])
            return carry
        lax.fori_loop(0, LSEQ, build_ts, 0)
        for s in range(3):
            for k in range(nk):
                ts_v[LSEQ * 3 + s, pl.ds(k * 16, 16)] = (
                    sen_v[s, pl.ds(k * 16, 16)])

        pltpu.sync_copy(tok_hbm.at[pl.ds(base, C)], idx_b[0])
        pltpu.sync_copy(mrk_hbm.at[pl.ds(base, C)], mrk_b[0])
        pltpu.async_copy(tbl_hbm.at[idx_b[0]], rows_b[0], gs_b[0])
        pltpu.async_copy(tok_hbm.at[pl.ds(base + C, C)], idx_b[1], is_b[1])
        pltpu.async_copy(mrk_hbm.at[pl.ds(base + C, C)], mrk_b[1], is_b[1])

        def compute_chunk(off, rows_v, idx_v, mrk_v):
            @plsc.parallel_loop(0, C, step=16)
            def group(r0):
                ids_g = idx_v[pl.ds(r0, 16)]
                mrk_g = mrk_v[pl.ds(r0, 16)]
                for j in range(16):
                    r = r0 + j
                    idj = ids_g[j]
                    mkj = mrk_g[j]
                    sj = jnp.where(mkj == 3, 0, mkj)
                    lj = (off + r) % LSEQ
                    tsi = jnp.where(idj != 0, lj, LSEQ) * 3 + sj
                    x = []
                    for k in range(nk):
                        tok_k = rows_v[r, pl.ds(k * 16, 16)]
                        ts_k = ts_v[tsi, pl.ds(k * 16, 16)]
                        x.append(tok_k + ts_k)
                    s1 = jnp.sum((x[0] + x[1]) + (x[2] + x[3]))
                    s2 = jnp.sum((x[0] * x[0] + x[1] * x[1])
                                 + (x[2] * x[2] + x[3] * x[3]))
                    mu = jnp.broadcast_to(s1, (16,)) * (1.0 / EMB)
                    var = jnp.broadcast_to(s2, (16,)) * (1.0 / EMB) - mu * mu
                    rs = _rsqrt(var + 1e-5)
                    for k in range(nk):
                        y = (x[k] - mu) * rs * g_k[k] + b_k[k]
                        rows_v[r, pl.ds(k * 16, 16)] = y

        def iter_body(i2, carry):
            for b in range(2):
                i = i2 * 2 + b
                off = base + i * C
                nb = 1 - b

                @pl.when(i + 1 < nchunks)
                def _():
                    offn = off + C

                    @pl.when(i >= 1)
                    def _():
                        pltpu.make_async_copy(
                            rows_b[nb], out_hbm.at[pl.ds(offn - 2 * C, C)],
                            os_b[nb]).wait()

                    pltpu.make_async_copy(
                        tok_hbm.at[pl.ds(offn, C)], idx_b[nb],
                        is_b[nb]).wait()
                    pltpu.make_async_copy(
                        mrk_hbm.at[pl.ds(offn, C)], mrk_b[nb],
                        is_b[nb]).wait()
                    pltpu.async_copy(tbl_hbm.at[idx_b[nb]], rows_b[nb],
                                     gs_b[nb])

                pltpu.make_async_copy(tbl_hbm.at[idx_b[b]], rows_b[b],
                                      gs_b[b]).wait()

                compute_chunk(off, rows_b[b], idx_b[b], mrk_b[b])
                pltpu.async_copy(rows_b[b], out_hbm.at[pl.ds(off, C)],
                                 os_b[b])

                @pl.when(i + 2 < nchunks)
                def _():
                    offn2 = off + 2 * C
                    pltpu.async_copy(tok_hbm.at[pl.ds(offn2, C)], idx_b[b],
                                     is_b[b])
                    pltpu.async_copy(mrk_hbm.at[pl.ds(offn2, C)], mrk_b[b],
                                     is_b[b])
            return carry

        lax.fori_loop(0, nchunks // 2, iter_body, 0)
        pltpu.make_async_copy(
            rows_b[0], out_hbm.at[pl.ds(base + (nchunks - 2) * C, C)],
            os_b[0]).wait()
        pltpu.make_async_copy(
            rows_b[1], out_hbm.at[pl.ds(base + (nchunks - 1) * C, C)],
            os_b[1]).wait()

    return body


def kernel(batTok, tokMrk, tokEmbTbl, timEmbTbl, senEmbTbl, gamma, beta):
    B, L = batTok.shape
    N = B * L
    tok_flat = batTok.reshape(N).astype(jnp.int32)
    mrk_flat = tokMrk.reshape(N).astype(jnp.int32)
    out = _make_kernel(N)(
        tok_flat, mrk_flat,
        tokEmbTbl.astype(jnp.float32),
        timEmbTbl.astype(jnp.float32),
        senEmbTbl.astype(jnp.float32),
        gamma.astype(jnp.float32),
        beta.astype(jnp.float32),
    )
    return out.reshape(B, L, EMB)
